# baseline (speedup 1.0000x reference)
import functools
import math

import jax
import jax.numpy as jnp
from jax.experimental import pallas as pl
from jax.experimental.pallas import tpu as pltpu


def _gather_kernel(ids_ref, table_ref, out_hbm, idx_smem, buf_a, buf_b,
                   sem_a, sem_b, isem, *, scale, unroll, chunks, npc):
    c = pl.program_id(0)
    i = pl.program_id(1)

    copy = pltpu.make_async_copy(ids_ref.at[0, 0], idx_smem, isem)
    copy.start()
    copy.wait()

    step_row0 = (c * npc + i) * chunks * unroll

    def run_chunk(k, parity, buf_ref, sem):
        ch = 2 * k + parity
        g = i * chunks + ch
        @pl.when(g >= 2)
        def _():
            pltpu.make_async_copy(buf_ref, out_hbm.at[pl.ds(0, 2 * unroll)],
                                  sem).wait()
        base = ch * unroll
        for u in range(unroll):
            w = table_ref[idx_smem[base + u], 0]
            wb = pltpu.bitcast(w.reshape(1, 128), jnp.bfloat16)
            buf_ref[2 * u:2 * u + 2, :] = wb.astype(jnp.float32) * scale
        pltpu.make_async_copy(
            buf_ref, out_hbm.at[pl.ds(2 * (step_row0 + base), 2 * unroll)],
            sem).start()

    def pair_body(k, carry):
        run_chunk(k, 0, buf_a, sem_a)
        run_chunk(k, 1, buf_b, sem_b)
        return carry

    jax.lax.fori_loop(0, chunks // 2, pair_body, 0)

    @pl.when(i == npc - 1)
    def _():
        pltpu.make_async_copy(buf_a, out_hbm.at[pl.ds(0, 2 * unroll)],
                              sem_a).wait()
        pltpu.make_async_copy(buf_b, out_hbm.at[pl.ds(0, 2 * unroll)],
                              sem_b).wait()


def kernel(ids, table):
    B, S = ids.shape
    V, D = table.shape
    scale = float(math.sqrt(D))

    n_tok = B * S
    TB = 16384
    U = 2048
    CHUNKS = TB // U

    step_tokens = 2 * TB
    n_pad = ((n_tok + step_tokens - 1) // step_tokens) * step_tokens
    flat_ids = ids.reshape(-1).astype(jnp.int32)
    if n_pad != n_tok:
        flat_ids = jnp.pad(flat_ids, (0, n_pad - n_tok))
    n_steps = n_pad // TB
    npc = n_steps // 2

    ids_3d = flat_ids.reshape(n_steps, 1, TB)
    table_bf = table.astype(jnp.bfloat16)
    table_i32 = jax.lax.bitcast_convert_type(
        table_bf.reshape(V, 1, 2, 128).transpose(0, 1, 3, 2), jnp.int32)

    out_flat = pl.pallas_call(
        functools.partial(_gather_kernel, scale=scale, unroll=U,
                          chunks=CHUNKS, npc=npc),
        out_shape=jax.ShapeDtypeStruct((n_pad * 2, 128), table.dtype),
        grid=(2, npc),
        in_specs=[
            pl.BlockSpec((1, 1, TB), lambda c, i: (c * npc + i, 0, 0)),
            pl.BlockSpec((V, 1, 128), lambda c, i: (0, 0, 0)),
        ],
        out_specs=pl.BlockSpec(memory_space=pl.ANY),
        scratch_shapes=[
            pltpu.SMEM((TB,), jnp.int32),
            pltpu.VMEM((U * 2, 128), jnp.float32),
            pltpu.VMEM((U * 2, 128), jnp.float32),
            pltpu.SemaphoreType.DMA,
            pltpu.SemaphoreType.DMA,
            pltpu.SemaphoreType.DMA,
        ],
        compiler_params=pltpu.CompilerParams(
            dimension_semantics=("parallel", "arbitrary"),
        ),
    )(ids_3d, table_i32)

    return out_flat[:2 * n_tok].reshape(B, S, D)


# speedup vs baseline: 1.8373x; 1.8373x over previous
import functools
import math

import jax
import jax.numpy as jnp
from jax.experimental import pallas as pl
from jax.experimental.pallas import tpu as pltpu


def _gather_kernel(ids_ref, table_ref, out_hbm, idx_smem, buf_a, buf_b,
                   sem_a, sem_b, isem, *, scale, unroll, chunks, n_steps):
    i = pl.program_id(0)

    copy = pltpu.make_async_copy(ids_ref.at[0, 0], idx_smem, isem)
    copy.start()
    copy.wait()

    for ch in range(chunks):
        buf, sem = (buf_a, sem_a) if ch % 2 == 0 else (buf_b, sem_b)
        if ch >= 2:
            pltpu.make_async_copy(buf, out_hbm.at[pl.ds(0, unroll)],
                                  sem).wait()
        else:
            @pl.when(i >= 1)
            def _(buf=buf, sem=sem):
                pltpu.make_async_copy(buf, out_hbm.at[pl.ds(0, unroll)],
                                      sem).wait()
        base = ch * unroll
        for u in range(unroll):
            buf[u, :] = table_ref[idx_smem[base + u], 0] * scale
        pltpu.make_async_copy(
            buf, out_hbm.at[pl.ds((i * chunks + ch) * unroll, unroll)],
            sem).start()

    @pl.when(i == n_steps - 1)
    def _():
        pltpu.make_async_copy(buf_a, out_hbm.at[pl.ds(0, unroll)],
                              sem_a).wait()
        pltpu.make_async_copy(buf_b, out_hbm.at[pl.ds(0, unroll)],
                              sem_b).wait()


def kernel(ids, table):
    B, S = ids.shape
    V, D = table.shape
    scale = float(math.sqrt(D))

    n_tok = B * S
    TB = 8192
    U = 1024
    CHUNKS = TB // U

    n_pad = ((n_tok + TB - 1) // TB) * TB
    flat_ids = ids.reshape(-1).astype(jnp.int32)
    if n_pad != n_tok:
        flat_ids = jnp.pad(flat_ids, (0, n_pad - n_tok))
    n_steps = n_pad // TB

    ids_3d = flat_ids.reshape(n_steps, 1, TB)
    table_3d = table.reshape(V, 1, D)

    out_flat = pl.pallas_call(
        functools.partial(_gather_kernel, scale=scale, unroll=U,
                          chunks=CHUNKS, n_steps=n_steps),
        out_shape=jax.ShapeDtypeStruct((n_pad, D), table.dtype),
        grid=(n_steps,),
        in_specs=[
            pl.BlockSpec((1, 1, TB), lambda i: (i, 0, 0)),
            pl.BlockSpec((V, 1, D), lambda i: (0, 0, 0)),
        ],
        out_specs=pl.BlockSpec(memory_space=pl.ANY),
        scratch_shapes=[
            pltpu.SMEM((TB,), jnp.int32),
            pltpu.VMEM((U, D), jnp.float32),
            pltpu.VMEM((U, D), jnp.float32),
            pltpu.SemaphoreType.DMA,
            pltpu.SemaphoreType.DMA,
            pltpu.SemaphoreType.DMA,
        ],
        compiler_params=pltpu.CompilerParams(
            dimension_semantics=("arbitrary",),
        ),
    )(ids_3d, table_3d)

    return out_flat[:n_tok].reshape(B, S, D)


# speedup vs baseline: 1.8395x; 1.0012x over previous
import functools
import math

import jax
import jax.numpy as jnp
from jax.experimental import pallas as pl
from jax.experimental.pallas import tpu as pltpu


def _gather_kernel(ids_ref, table_ref, out_hbm, idx_smem, buf_a, buf_b,
                   sem_a, sem_b, isem, *, scale, unroll, chunks, n_steps):
    i = pl.program_id(0)

    copy = pltpu.make_async_copy(ids_ref.at[0, 0], idx_smem, isem)
    copy.start()
    copy.wait()

    for ch in range(chunks):
        buf, sem = (buf_a, sem_a) if ch % 2 == 0 else (buf_b, sem_b)
        if ch >= 2:
            pltpu.make_async_copy(buf, out_hbm.at[pl.ds(0, unroll)],
                                  sem).wait()
        else:
            @pl.when(i >= 1)
            def _(buf=buf, sem=sem):
                pltpu.make_async_copy(buf, out_hbm.at[pl.ds(0, unroll)],
                                      sem).wait()
        base = ch * unroll
        for u in range(unroll):
            buf[u, :] = table_ref[idx_smem[base + u], 0] * scale
        pltpu.make_async_copy(
            buf, out_hbm.at[pl.ds((i * chunks + ch) * unroll, unroll)],
            sem).start()

    @pl.when(i == n_steps - 1)
    def _():
        pltpu.make_async_copy(buf_a, out_hbm.at[pl.ds(0, unroll)],
                              sem_a).wait()
        pltpu.make_async_copy(buf_b, out_hbm.at[pl.ds(0, unroll)],
                              sem_b).wait()


def kernel(ids, table):
    B, S = ids.shape
    V, D = table.shape
    scale = float(math.sqrt(D))

    n_tok = B * S
    TB = 8192
    U = 2048
    CHUNKS = TB // U

    n_pad = ((n_tok + TB - 1) // TB) * TB
    flat_ids = ids.reshape(-1).astype(jnp.int32)
    if n_pad != n_tok:
        flat_ids = jnp.pad(flat_ids, (0, n_pad - n_tok))
    n_steps = n_pad // TB

    ids_3d = flat_ids.reshape(n_steps, 1, TB)
    table_3d = table.reshape(V, 1, D)

    out_flat = pl.pallas_call(
        functools.partial(_gather_kernel, scale=scale, unroll=U,
                          chunks=CHUNKS, n_steps=n_steps),
        out_shape=jax.ShapeDtypeStruct((n_pad, D), table.dtype),
        grid=(n_steps,),
        in_specs=[
            pl.BlockSpec((1, 1, TB), lambda i: (i, 0, 0)),
            pl.BlockSpec((V, 1, D), lambda i: (0, 0, 0)),
        ],
        out_specs=pl.BlockSpec(memory_space=pl.ANY),
        scratch_shapes=[
            pltpu.SMEM((TB,), jnp.int32),
            pltpu.VMEM((U, D), jnp.float32),
            pltpu.VMEM((U, D), jnp.float32),
            pltpu.SemaphoreType.DMA,
            pltpu.SemaphoreType.DMA,
            pltpu.SemaphoreType.DMA,
        ],
        compiler_params=pltpu.CompilerParams(
            dimension_semantics=("arbitrary",),
        ),
    )(ids_3d, table_3d)

    return out_flat[:n_tok].reshape(B, S, D)


# speedup vs baseline: 1.8894x; 1.0271x over previous
import functools
import math

import jax
import jax.numpy as jnp
from jax.experimental import pallas as pl
from jax.experimental.pallas import tpu as pltpu


def _gather_kernel(ids_ref, table_ref, out_hbm, idx_smem, buf_a, buf_b,
                   sem_a, sem_b, isem, *, scale, unroll, chunks, n_steps):
    i = pl.program_id(0)

    copy = pltpu.make_async_copy(ids_ref.at[0, 0], idx_smem, isem)
    copy.start()
    copy.wait()

    for ch in range(chunks):
        buf, sem = (buf_a, sem_a) if ch % 2 == 0 else (buf_b, sem_b)
        if ch >= 2:
            pltpu.make_async_copy(buf, out_hbm.at[pl.ds(0, unroll)],
                                  sem).wait()
        else:
            @pl.when(i >= 1)
            def _(buf=buf, sem=sem):
                pltpu.make_async_copy(buf, out_hbm.at[pl.ds(0, unroll)],
                                      sem).wait()
        base = ch * unroll
        for u in range(unroll):
            buf[u, :] = table_ref[idx_smem[base + u], 0] * scale
        pltpu.make_async_copy(
            buf, out_hbm.at[pl.ds((i * chunks + ch) * unroll, unroll)],
            sem).start()

    @pl.when(i == n_steps - 1)
    def _():
        pltpu.make_async_copy(buf_a, out_hbm.at[pl.ds(0, unroll)],
                              sem_a).wait()
        pltpu.make_async_copy(buf_b, out_hbm.at[pl.ds(0, unroll)],
                              sem_b).wait()


def kernel(ids, table):
    B, S = ids.shape
    V, D = table.shape
    scale = float(math.sqrt(D))

    n_tok = B * S
    TB = 16384
    U = 2048
    CHUNKS = TB // U

    n_pad = ((n_tok + TB - 1) // TB) * TB
    flat_ids = ids.reshape(-1).astype(jnp.int32)
    if n_pad != n_tok:
        flat_ids = jnp.pad(flat_ids, (0, n_pad - n_tok))
    n_steps = n_pad // TB

    ids_3d = flat_ids.reshape(n_steps, 1, TB)
    table_3d = table.reshape(V, 1, D)

    out_flat = pl.pallas_call(
        functools.partial(_gather_kernel, scale=scale, unroll=U,
                          chunks=CHUNKS, n_steps=n_steps),
        out_shape=jax.ShapeDtypeStruct((n_pad, D), table.dtype),
        grid=(n_steps,),
        in_specs=[
            pl.BlockSpec((1, 1, TB), lambda i: (i, 0, 0)),
            pl.BlockSpec((V, 1, D), lambda i: (0, 0, 0)),
        ],
        out_specs=pl.BlockSpec(memory_space=pl.ANY),
        scratch_shapes=[
            pltpu.SMEM((TB,), jnp.int32),
            pltpu.VMEM((U, D), jnp.float32),
            pltpu.VMEM((U, D), jnp.float32),
            pltpu.SemaphoreType.DMA,
            pltpu.SemaphoreType.DMA,
            pltpu.SemaphoreType.DMA,
        ],
        compiler_params=pltpu.CompilerParams(
            dimension_semantics=("arbitrary",),
        ),
    )(ids_3d, table_3d)

    return out_flat[:n_tok].reshape(B, S, D)


# speedup vs baseline: 1.8996x; 1.0054x over previous
import functools
import math

import jax
import jax.numpy as jnp
from jax.experimental import pallas as pl
from jax.experimental.pallas import tpu as pltpu


def _gather_kernel(ids_ref, table_ref, out_hbm, idx_smem, buf_a, buf_b,
                   sem_a, sem_b, isem, *, scale, unroll, chunks, n_steps):
    i = pl.program_id(0)

    copy = pltpu.make_async_copy(ids_ref.at[0, 0], idx_smem, isem)
    copy.start()
    copy.wait()

    for ch in range(chunks):
        buf, sem = (buf_a, sem_a) if ch % 2 == 0 else (buf_b, sem_b)
        if ch >= 2:
            pltpu.make_async_copy(buf, out_hbm.at[pl.ds(0, unroll)],
                                  sem).wait()
        else:
            @pl.when(i >= 1)
            def _(buf=buf, sem=sem):
                pltpu.make_async_copy(buf, out_hbm.at[pl.ds(0, unroll)],
                                      sem).wait()
        base = ch * unroll
        for u in range(unroll):
            buf[u, :] = table_ref[idx_smem[base + u], 0] * scale
        pltpu.make_async_copy(
            buf, out_hbm.at[pl.ds((i * chunks + ch) * unroll, unroll)],
            sem).start()

    @pl.when(i == n_steps - 1)
    def _():
        pltpu.make_async_copy(buf_a, out_hbm.at[pl.ds(0, unroll)],
                              sem_a).wait()
        pltpu.make_async_copy(buf_b, out_hbm.at[pl.ds(0, unroll)],
                              sem_b).wait()


def kernel(ids, table):
    B, S = ids.shape
    V, D = table.shape
    scale = float(math.sqrt(D))

    n_tok = B * S
    TB = 32768
    U = 2048
    CHUNKS = TB // U

    n_pad = ((n_tok + TB - 1) // TB) * TB
    flat_ids = ids.reshape(-1).astype(jnp.int32)
    if n_pad != n_tok:
        flat_ids = jnp.pad(flat_ids, (0, n_pad - n_tok))
    n_steps = n_pad // TB

    ids_3d = flat_ids.reshape(n_steps, 1, TB)
    table_3d = table.reshape(V, 1, D)

    out_flat = pl.pallas_call(
        functools.partial(_gather_kernel, scale=scale, unroll=U,
                          chunks=CHUNKS, n_steps=n_steps),
        out_shape=jax.ShapeDtypeStruct((n_pad, D), table.dtype),
        grid=(n_steps,),
        in_specs=[
            pl.BlockSpec((1, 1, TB), lambda i: (i, 0, 0)),
            pl.BlockSpec((V, 1, D), lambda i: (0, 0, 0)),
        ],
        out_specs=pl.BlockSpec(memory_space=pl.ANY),
        scratch_shapes=[
            pltpu.SMEM((TB,), jnp.int32),
            pltpu.VMEM((U, D), jnp.float32),
            pltpu.VMEM((U, D), jnp.float32),
            pltpu.SemaphoreType.DMA,
            pltpu.SemaphoreType.DMA,
            pltpu.SemaphoreType.DMA,
        ],
        compiler_params=pltpu.CompilerParams(
            dimension_semantics=("arbitrary",),
        ),
    )(ids_3d, table_3d)

    return out_flat[:n_tok].reshape(B, S, D)


# speedup vs baseline: 1.9021x; 1.0013x over previous
import functools
import math

import jax
import jax.numpy as jnp
from jax.experimental import pallas as pl
from jax.experimental.pallas import tpu as pltpu


def _gather_kernel(ids_ref, table_ref, out_hbm, idx_smem, buf_a, buf_b,
                   buf_c, buf_d, sem_a, sem_b, sem_c, sem_d, isem, *,
                   scale, unroll, chunks, n_steps):
    i = pl.program_id(0)

    copy = pltpu.make_async_copy(ids_ref.at[0, 0], idx_smem, isem)
    copy.start()
    copy.wait()

    bufs = [(buf_a, sem_a), (buf_b, sem_b), (buf_c, sem_c), (buf_d, sem_d)]
    for ch in range(chunks):
        buf, sem = bufs[ch % 4]
        if ch >= 4:
            pltpu.make_async_copy(buf, out_hbm.at[pl.ds(0, unroll)],
                                  sem).wait()
        else:
            @pl.when(i >= 1)
            def _(buf=buf, sem=sem):
                pltpu.make_async_copy(buf, out_hbm.at[pl.ds(0, unroll)],
                                      sem).wait()
        base = ch * unroll
        for u in range(unroll):
            buf[u, :] = table_ref[idx_smem[base + u], 0] * scale
        pltpu.make_async_copy(
            buf, out_hbm.at[pl.ds((i * chunks + ch) * unroll, unroll)],
            sem).start()

    @pl.when(i == n_steps - 1)
    def _():
        for b, s in [(buf_a, sem_a), (buf_b, sem_b), (buf_c, sem_c),
                     (buf_d, sem_d)]:
            pltpu.make_async_copy(b, out_hbm.at[pl.ds(0, unroll)],
                                  s).wait()


def kernel(ids, table):
    B, S = ids.shape
    V, D = table.shape
    scale = float(math.sqrt(D))

    n_tok = B * S
    TB = 32768
    U = 2048
    CHUNKS = TB // U

    n_pad = ((n_tok + TB - 1) // TB) * TB
    flat_ids = ids.reshape(-1).astype(jnp.int32)
    if n_pad != n_tok:
        flat_ids = jnp.pad(flat_ids, (0, n_pad - n_tok))
    n_steps = n_pad // TB

    ids_3d = flat_ids.reshape(n_steps, 1, TB)
    table_3d = table.reshape(V, 1, D)

    out_flat = pl.pallas_call(
        functools.partial(_gather_kernel, scale=scale, unroll=U,
                          chunks=CHUNKS, n_steps=n_steps),
        out_shape=jax.ShapeDtypeStruct((n_pad, D), table.dtype),
        grid=(n_steps,),
        in_specs=[
            pl.BlockSpec((1, 1, TB), lambda i: (i, 0, 0)),
            pl.BlockSpec((V, 1, D), lambda i: (0, 0, 0)),
        ],
        out_specs=pl.BlockSpec(memory_space=pl.ANY),
        scratch_shapes=[
            pltpu.SMEM((TB,), jnp.int32),
            pltpu.VMEM((U, D), jnp.float32),
            pltpu.VMEM((U, D), jnp.float32),
            pltpu.VMEM((U, D), jnp.float32),
            pltpu.VMEM((U, D), jnp.float32),
            pltpu.SemaphoreType.DMA,
            pltpu.SemaphoreType.DMA,
            pltpu.SemaphoreType.DMA,
            pltpu.SemaphoreType.DMA,
            pltpu.SemaphoreType.DMA,
        ],
        compiler_params=pltpu.CompilerParams(
            dimension_semantics=("arbitrary",),
        ),
    )(ids_3d, table_3d)

    return out_flat[:n_tok].reshape(B, S, D)


# speedup vs baseline: 1.9887x; 1.0455x over previous
import functools
import math

import jax
import jax.numpy as jnp
from jax.experimental import pallas as pl
from jax.experimental.pallas import tpu as pltpu


def _gather_kernel(ids_ref, table_ref, out_hbm, idx_smem, buf_a, buf_b,
                   buf_c, buf_d, sem_a, sem_b, sem_c, sem_d, isem_a,
                   isem_b, *, scale, unroll, chunks, n_steps):
    i = pl.program_id(0)

    tb = chunks * unroll
    q = tb // 4
    q_chunks = chunks // 4

    def ids_copy(qi, qsem):
        return pltpu.make_async_copy(ids_ref.at[0, 0, pl.ds(qi * q, q)],
                                     idx_smem.at[pl.ds(qi * q, q)], qsem)

    ids_copy(0, isem_a).start()
    ids_copy(1, isem_b).start()

    bufs = [(buf_a, sem_a), (buf_b, sem_b), (buf_c, sem_c), (buf_d, sem_d)]
    for ch in range(chunks):
        if ch % q_chunks == 0:
            qi = ch // q_chunks
            qsem = isem_a if qi % 2 == 0 else isem_b
            ids_copy(qi, qsem).wait()
            if qi + 2 < 4:
                ids_copy(qi + 2, qsem).start()
        buf, sem = bufs[ch % 4]
        if ch >= 4:
            pltpu.make_async_copy(buf, out_hbm.at[pl.ds(0, unroll)],
                                  sem).wait()
        else:
            @pl.when(i >= 1)
            def _(buf=buf, sem=sem):
                pltpu.make_async_copy(buf, out_hbm.at[pl.ds(0, unroll)],
                                      sem).wait()
        base = ch * unroll
        for u in range(unroll):
            buf[u, :] = table_ref[idx_smem[base + u], 0] * scale
        pltpu.make_async_copy(
            buf, out_hbm.at[pl.ds((i * chunks + ch) * unroll, unroll)],
            sem).start()

    @pl.when(i == n_steps - 1)
    def _():
        for b, s in [(buf_a, sem_a), (buf_b, sem_b), (buf_c, sem_c),
                     (buf_d, sem_d)]:
            pltpu.make_async_copy(b, out_hbm.at[pl.ds(0, unroll)],
                                  s).wait()


def kernel(ids, table):
    B, S = ids.shape
    V, D = table.shape
    scale = float(math.sqrt(D))

    n_tok = B * S
    TB = 32768
    U = 2048
    CHUNKS = TB // U

    n_pad = ((n_tok + TB - 1) // TB) * TB
    flat_ids = ids.reshape(-1).astype(jnp.int32)
    if n_pad != n_tok:
        flat_ids = jnp.pad(flat_ids, (0, n_pad - n_tok))
    n_steps = n_pad // TB

    ids_3d = flat_ids.reshape(n_steps, 1, TB)
    table_3d = table.reshape(V, 1, D)

    out_flat = pl.pallas_call(
        functools.partial(_gather_kernel, scale=scale, unroll=U,
                          chunks=CHUNKS, n_steps=n_steps),
        out_shape=jax.ShapeDtypeStruct((n_pad, D), table.dtype),
        grid=(n_steps,),
        in_specs=[
            pl.BlockSpec((1, 1, TB), lambda i: (i, 0, 0)),
            pl.BlockSpec((V, 1, D), lambda i: (0, 0, 0)),
        ],
        out_specs=pl.BlockSpec(memory_space=pl.ANY),
        scratch_shapes=[
            pltpu.SMEM((TB,), jnp.int32),
            pltpu.VMEM((U, D), jnp.float32),
            pltpu.VMEM((U, D), jnp.float32),
            pltpu.VMEM((U, D), jnp.float32),
            pltpu.VMEM((U, D), jnp.float32),
            pltpu.SemaphoreType.DMA,
            pltpu.SemaphoreType.DMA,
            pltpu.SemaphoreType.DMA,
            pltpu.SemaphoreType.DMA,
            pltpu.SemaphoreType.DMA,
            pltpu.SemaphoreType.DMA,
        ],
        compiler_params=pltpu.CompilerParams(
            dimension_semantics=("arbitrary",),
        ),
    )(ids_3d, table_3d)

    return out_flat[:n_tok].reshape(B, S, D)
